# Initial kernel scaffold; baseline (speedup 1.0000x reference)
import jax
import jax.numpy as jnp
from jax.experimental import pallas as pl


def kernel(image_features, expression, position, cell_type, xemb, yemb, ct_emb, ct_W, ct_b, ct_g, ct_be, ip_pW, ip_pb, ip_fW, ip_fb, ip_g, ip_b, dh_W, dh_b, dt_W, dt_b, sp_W, sp_b, dg_W, dg_as, dg_ad, dg_bias, sg_W, sg_as, sg_ad, sg_bias, fu_W, fu_b, fu_g, fu_be, spp_pW, spp_pb, spp_fW, spp_fb, spp_g, spp_b):
    raise NotImplementedError("write your pallas kernel here")


# speedup vs baseline: 1.1834x; 1.1834x over previous
import functools

import jax
import jax.numpy as jnp
from jax import lax
from jax.experimental import pallas as pl
from jax.experimental.pallas import tpu as pltpu
from jax.experimental.pallas import tpu_sc as plsc

N = 4096
SPOT = 256
IMG = 1024
P = 128
H = 4
K = 5
RB = 512
NW = 32
RW = N // NW
GB = N // RB

F32 = jnp.float32
I32 = jnp.int32


def _ln(x, g, b):
    m = jnp.mean(x, axis=-1, keepdims=True)
    v = jnp.mean((x - m) * (x - m), axis=-1, keepdims=True)
    return (x - m) / jnp.sqrt(v + 1e-5) * g + b


def _sc_gather(xemb, yemb, xi, yi):
    k = functools.partial(
        pl.kernel,
        out_type=jax.ShapeDtypeStruct((N, SPOT), F32),
        mesh=plsc.VectorSubcoreMesh(core_axis_name="c", subcore_axis_name="s"),
        scratch_types=[
            pltpu.VMEM((RW,), I32),
            pltpu.VMEM((RW,), I32),
            pltpu.VMEM((RW, SPOT), F32),
            pltpu.VMEM((RW, SPOT), F32),
            pltpu.SemaphoreType.DMA,
            pltpu.SemaphoreType.DMA,
        ],
    )(_sc_gather_body)
    return k(xemb, yemb, xi, yi)


def _sc_gather_body(xemb, yemb, xi, yi, out, ix, iy, rx, ry, s1, s2):
    c = lax.axis_index("c")
    s = lax.axis_index("s")
    base = (s * 2 + c) * RW
    pltpu.sync_copy(xi.at[pl.ds(base, RW)], ix)
    pltpu.sync_copy(yi.at[pl.ds(base, RW)], iy)
    cp1 = pltpu.async_copy(xemb.at[ix], rx, s1)
    cp2 = pltpu.async_copy(yemb.at[iy], ry, s2)
    cp1.wait()
    cp2.wait()

    def arow(r, _):
        for q in range(SPOT // 16):
            sl = pl.ds(q * 16, 16)
            rx[r, sl] = rx[r, sl] + ry[r, sl]
        return 0

    lax.fori_loop(0, RW, arow, 0)
    pltpu.sync_copy(rx, out.at[pl.ds(base, RW)])


def _dense_body(ctpos, cxy, expr, ct_emb, ct_W, ct_b, ct_g, ct_be,
                dh_W, dh_b, dt_W, dt_b, sp_W, sp_b,
                dg_W, dg_a, sg_W, sg_a,
                head_o, tail_o, hdyn_o, hspa_o, adyn_o, aspa_o):
    ct = ctpos[:, 2:3]
    tbl = jnp.dot(ct_emb[...], ct_W[...], preferred_element_type=F32) + ct_b[...]
    tbl = jnp.maximum(tbl, 0.0)
    tbl = _ln(tbl, ct_g[...], ct_be[...])
    oh = (ct == lax.broadcasted_iota(I32, (1, 8), 1).astype(F32)).astype(F32)
    cte = jnp.dot(oh, tbl, preferred_element_type=F32)
    spot = cxy[...] + expr[...] + cte
    head = jnp.dot(spot, dh_W[...], preferred_element_type=F32) + dh_b[...]
    tail = jnp.dot(spot, dt_W[...], preferred_element_type=F32) + dt_b[...]
    spa = jnp.dot(spot, sp_W[...], preferred_element_type=F32) + sp_b[...]
    head_o[...] = head
    tail_o[...] = tail
    ind = (lax.broadcasted_iota(I32, (H * P, H), 0) // P
           == lax.broadcasted_iota(I32, (H * P, H), 1)).astype(F32)
    for x, W, a, h_o, a_o in ((head, dg_W, dg_a, hdyn_o, adyn_o),
                              (spa, sg_W, sg_a, hspa_o, aspa_o)):
        h = jnp.dot(x, W[...], preferred_element_type=F32)
        h_o[...] = h
        asrc = jnp.dot(h * a[0:1, :], ind, preferred_element_type=F32)
        adst = jnp.dot(h * a[1:2, :], ind, preferred_element_type=F32)
        a_o[...] = jnp.concatenate([asrc, adst], axis=1)


def _tc_dense(ctpos, cxy, expr, ct_emb8, ct_W, ct_b, ct_g, ct_be,
              dh_W, dh_b, dt_W, dt_b, sp_W, sp_b, dg_W, dg_a, sg_W, sg_a):
    row = lambda i: (i, 0)
    full = lambda i: (0, 0)
    sd = jax.ShapeDtypeStruct
    return pl.pallas_call(
        _dense_body,
        grid=(GB,),
        in_specs=[
            pl.BlockSpec((RB, 8), row), pl.BlockSpec((RB, SPOT), row),
            pl.BlockSpec((RB, SPOT), row),
            pl.BlockSpec((8, P), full), pl.BlockSpec((P, SPOT), full),
            pl.BlockSpec((1, SPOT), full), pl.BlockSpec((1, SPOT), full),
            pl.BlockSpec((1, SPOT), full),
            pl.BlockSpec((SPOT, P), full), pl.BlockSpec((1, P), full),
            pl.BlockSpec((SPOT, P), full), pl.BlockSpec((1, P), full),
            pl.BlockSpec((SPOT, P), full), pl.BlockSpec((1, P), full),
            pl.BlockSpec((P, H * P), full), pl.BlockSpec((2, H * P), full),
            pl.BlockSpec((P, H * P), full), pl.BlockSpec((2, H * P), full),
        ],
        out_specs=[
            pl.BlockSpec((RB, P), row), pl.BlockSpec((RB, P), row),
            pl.BlockSpec((RB, H * P), row), pl.BlockSpec((RB, H * P), row),
            pl.BlockSpec((RB, 8), row), pl.BlockSpec((RB, 8), row),
        ],
        out_shape=[sd((N, P), F32), sd((N, P), F32),
                   sd((N, H * P), F32), sd((N, H * P), F32),
                   sd((N, 8), F32), sd((N, 8), F32)],
    )(ctpos, cxy, expr, ct_emb8, ct_W, ct_b, ct_g, ct_be,
      dh_W, dh_b, dt_W, dt_b, sp_W, sp_b, dg_W, dg_a, sg_W, sg_a)


def _top5(s, out_ref):
    cols = lax.broadcasted_iota(I32, s.shape, 1)
    for k in range(K):
        m = jnp.max(s, axis=1, keepdims=True)
        idx = jnp.min(jnp.where(s == m, cols, jnp.int32(2 ** 30)), axis=1)
        out_ref[k, :] = idx
        s = jnp.where(cols == idx[:, None], jnp.float32(-3e38), s)


def _topk_body(ctpos, ctposT, head, tailT, tdyn_o, tspa_o):
    fsim = jnp.dot(head[...], tailT[...], preferred_element_type=F32)
    csame = (ctpos[:, 2:3] == ctposT[2:3, :]).astype(F32)
    _top5(fsim + 0.3 * csame, tdyn_o)
    xr = ctpos[:, 0:1]
    yr = ctpos[:, 1:2]
    xc = ctposT[0:1, :]
    yc = ctposT[1:2, :]
    sqr = xr * xr + yr * yr
    sqc = xc * xc + yc * yc
    bxr = xr.astype(jnp.bfloat16).astype(F32)
    byr = yr.astype(jnp.bfloat16).astype(F32)
    bxc = xc.astype(jnp.bfloat16).astype(F32)
    byc = yc.astype(jnp.bfloat16).astype(F32)
    dot2 = bxr * bxc + byr * byc
    d2 = jnp.maximum(sqr + sqc - 2.0 * dot2, 0.0)
    _top5(-d2, tspa_o)


def _tc_topk(ctpos, ctposT, head, tailT):
    sd = jax.ShapeDtypeStruct
    return pl.pallas_call(
        _topk_body,
        grid=(GB,),
        in_specs=[
            pl.BlockSpec((RB, 8), lambda i: (i, 0)),
            pl.BlockSpec((8, N), lambda i: (0, 0)),
            pl.BlockSpec((RB, P), lambda i: (i, 0)),
            pl.BlockSpec((P, N), lambda i: (0, 0)),
        ],
        out_specs=[
            pl.BlockSpec((8, RB), lambda i: (0, i)),
            pl.BlockSpec((8, RB), lambda i: (0, i)),
        ],
        out_shape=[sd((8, N), I32), sd((8, N), I32)],
    )(ctpos, ctposT, head, tailT)


def _gat_jax(h, a2, T2, bias):
    hh = h.reshape(N, H, P)
    asrc, adst = a2[:, 0:4], a2[:, 4:8]
    src = jnp.concatenate([jnp.arange(N)] * (K + 1))
    dst = jnp.concatenate([T2[k] for k in range(K)] + [jnp.arange(N)])
    e = asrc[src] + adst[dst]
    ex = jnp.exp(jnp.maximum(e, 0.2 * e))
    den = jax.ops.segment_sum(ex, dst, num_segments=N)
    out = jax.ops.segment_sum(hh[src] * ex[:, :, None], dst, num_segments=N)
    out = out / den[:, :, None]
    return out.reshape(N, H * P) + bias


def _proj_head_blk(x, pW, pb, fW, fb, g, b):
    p = jnp.dot(x, pW[...], preferred_element_type=F32) + pb[...]
    h = jax.nn.gelu(p)
    h = jnp.dot(h, fW[...], preferred_element_type=F32) + fb[...]
    h = h + p
    return _ln(h, g[...], b[...])


def _img_body(x, pW, pb, fW, fb, g, b, out):
    out[...] = _proj_head_blk(x[...], pW, pb, fW, fb, g, b)


def _tc_img(image_features, ip_pW, ip_pb, ip_fW, ip_fb, ip_g, ip_b):
    full = lambda i: (0, 0)
    return pl.pallas_call(
        _img_body,
        grid=(GB,),
        in_specs=[
            pl.BlockSpec((RB, IMG), lambda i: (i, 0)),
            pl.BlockSpec((IMG, P), full), pl.BlockSpec((1, P), full),
            pl.BlockSpec((P, P), full), pl.BlockSpec((1, P), full),
            pl.BlockSpec((1, P), full), pl.BlockSpec((1, P), full),
        ],
        out_specs=pl.BlockSpec((RB, P), lambda i: (i, 0)),
        out_shape=jax.ShapeDtypeStruct((N, P), F32),
    )(image_features, ip_pW, ip_pb, ip_fW, ip_fb, ip_g, ip_b)


def _mid_body(gd, gs, fuW1, fuW2, fu_b, fu_g, fu_be,
              spW, spb, sfW, sfb, sg, sb, out):
    f = (jnp.dot(gd[...], fuW1[...], preferred_element_type=F32)
         + jnp.dot(gs[...], fuW2[...], preferred_element_type=F32) + fu_b[...])
    f = jnp.maximum(f, 0.0)
    f = _ln(f, fu_g[...], fu_be[...])
    out[...] = _proj_head_blk(f, spW, spb, sfW, sfb, sg, sb)


def _tc_mid(gat_dyn, gat_spa, fuW1, fuW2, fu_b, fu_g, fu_be,
            spp_pW, spp_pb, spp_fW, spp_fb, spp_g, spp_b):
    full = lambda i: (0, 0)
    return pl.pallas_call(
        _mid_body,
        grid=(GB,),
        in_specs=[
            pl.BlockSpec((RB, H * P), lambda i: (i, 0)),
            pl.BlockSpec((RB, H * P), lambda i: (i, 0)),
            pl.BlockSpec((H * P, P), full), pl.BlockSpec((H * P, P), full),
            pl.BlockSpec((1, P), full), pl.BlockSpec((1, P), full),
            pl.BlockSpec((1, P), full),
            pl.BlockSpec((P, P), full), pl.BlockSpec((1, P), full),
            pl.BlockSpec((P, P), full), pl.BlockSpec((1, P), full),
            pl.BlockSpec((1, P), full), pl.BlockSpec((1, P), full),
        ],
        out_specs=pl.BlockSpec((RB, P), lambda i: (i, 0)),
        out_shape=jax.ShapeDtypeStruct((N, P), F32),
    )(gat_dyn, gat_spa, fuW1, fuW2, fu_b, fu_g, fu_be,
      spp_pW, spp_pb, spp_fW, spp_fb, spp_g, spp_b)


def _loss_body(se, imgT, out, Mref, Sref, accref):
    i = pl.program_id(0)
    cos = jnp.dot(se[...], imgT[...], preferred_element_type=F32)
    cols = lax.broadcasted_iota(I32, (RB, N), 1)
    rows = lax.broadcasted_iota(I32, (RB, N), 0) + i * RB
    diag_sum = jnp.sum(jnp.where(cols == rows, cos, 0.0))
    m_r = jnp.max(cos, axis=1, keepdims=True)
    lse_r = jnp.sum(m_r[:, 0] + jnp.log(jnp.sum(jnp.exp(cos - m_r), axis=1)))

    @pl.when(i == 0)
    def _():
        Mref[...] = jnp.full((1, N), -3e38, F32)
        Sref[...] = jnp.zeros((1, N), F32)
        accref[0] = 0.0
        accref[1] = 0.0

    bm = jnp.max(cos, axis=0, keepdims=True)
    newM = jnp.maximum(Mref[...], bm)
    Sref[...] = (Sref[...] * jnp.exp(Mref[...] - newM)
                 + jnp.sum(jnp.exp(cos - newM), axis=0, keepdims=True))
    Mref[...] = newM
    accref[0] = accref[0] + diag_sum
    accref[1] = accref[1] + lse_r

    @pl.when(i == GB - 1)
    def _():
        col_lse = jnp.sum(Mref[...] + jnp.log(Sref[...]))
        val = 0.5 * (accref[1] / N + col_lse / N) - accref[0] / N
        out[...] = jnp.reshape(val, (1, 1))


def _tc_loss(se, imgT):
    return pl.pallas_call(
        _loss_body,
        grid=(GB,),
        in_specs=[
            pl.BlockSpec((RB, P), lambda i: (i, 0)),
            pl.BlockSpec((P, N), lambda i: (0, 0)),
        ],
        out_specs=pl.BlockSpec((1, 1), lambda i: (0, 0)),
        out_shape=jax.ShapeDtypeStruct((1, 1), F32),
        scratch_shapes=[
            pltpu.VMEM((1, N), F32),
            pltpu.VMEM((1, N), F32),
            pltpu.SMEM((2,), F32),
        ],
    )(se, imgT)


def kernel(image_features, expression, position, cell_type, xemb, yemb,
           ct_emb, ct_W, ct_b, ct_g, ct_be, ip_pW, ip_pb, ip_fW, ip_fb, ip_g,
           ip_b, dh_W, dh_b, dt_W, dt_b, sp_W, sp_b, dg_W, dg_as, dg_ad,
           dg_bias, sg_W, sg_as, sg_ad, sg_bias, fu_W, fu_b, fu_g, fu_be,
           spp_pW, spp_pb, spp_fW, spp_fb, spp_g, spp_b):
    xi = position[:, 0].astype(I32)
    yi = position[:, 1].astype(I32)
    ctf = cell_type.astype(F32)[:, None]
    ctpos = jnp.concatenate([position, ctf, jnp.zeros((N, 5), F32)], axis=1)
    ctposT = ctpos.T
    ct_emb8 = jnp.pad(ct_emb, ((0, 2), (0, 0)))
    dg_a = jnp.stack([dg_as.reshape(-1), dg_ad.reshape(-1)])
    sg_a = jnp.stack([sg_as.reshape(-1), sg_ad.reshape(-1)])
    r1 = lambda v: v.reshape(1, -1)

    cxy = _sc_gather(xemb, yemb, xi, yi)
    head, tail, hdyn, hspa, adyn, aspa = _tc_dense(
        ctpos, cxy, expression, ct_emb8, ct_W, r1(ct_b), r1(ct_g), r1(ct_be),
        dh_W, r1(dh_b), dt_W, r1(dt_b), sp_W, r1(sp_b), dg_W, dg_a, sg_W, sg_a)
    tdyn, tspa = _tc_topk(ctpos, ctposT, head, tail.T)

    gat_dyn = _gat_jax(hdyn, adyn, tdyn, dg_bias)
    gat_spa = _gat_jax(hspa, aspa, tspa, sg_bias)

    img_emb = _tc_img(image_features, ip_pW, r1(ip_pb), ip_fW, r1(ip_fb),
                      r1(ip_g), r1(ip_b))
    se = _tc_mid(gat_dyn, gat_spa, fu_W[:H * P], fu_W[H * P:], r1(fu_b),
                 r1(fu_g), r1(fu_be), spp_pW, r1(spp_pb), spp_fW, r1(spp_fb),
                 r1(spp_g), r1(spp_b))
    loss = _tc_loss(se, img_emb.T)
    return loss.reshape(())
